# Initial kernel scaffold; baseline (speedup 1.0000x reference)
import jax
import jax.numpy as jnp
from jax.experimental import pallas as pl


def kernel(dr_vec, Z, idx, params):
    raise NotImplementedError("write your pallas kernel here")


# speedup vs baseline: 1.3030x; 1.3030x over previous
import functools

import jax
import jax.numpy as jnp
import numpy as np
from jax import lax
from jax.experimental import pallas as pl
from jax.experimental.pallas import tpu as pltpu
from jax.experimental.pallas import tpu_sc as plsc

F = 128
K = 32
H = 4
DH = F // H
CUTOFF = 5.0
N = 10000
NPAD = 10240
E = 320000
EPAD = 323584
BN = 1024
GN = NPAD // BN
BE = 2048
GE = EPAD // BE
NC = 2
NS = 16
NW = NC * NS
CH = 128
EW = EPAD // NW
NCHUNK = EW // CH
ROWS_N = NPAD // NS

@functools.lru_cache(maxsize=1)
def _sc_mesh():
    return plsc.VectorSubcoreMesh(
        core_axis_name="c", subcore_axis_name="s",
        num_cores=NC, num_subcores=NS)

_f32 = jnp.float32


def _silu(x):
    return x / (1.0 + jnp.exp(-x))


def _node_proj_body(z_ref, emb_ref, wq_ref, wk_ref, wv_ref,
                    h_ref, q_ref, k_ref, v_ref):
    z = z_ref[...]
    col = lax.broadcasted_iota(jnp.int32, (BN, 128), 1)
    oh = (col == z).astype(_f32)
    h = jnp.dot(oh, emb_ref[...], preferred_element_type=_f32)
    h_ref[...] = h
    q_ref[...] = jnp.dot(h, wq_ref[...], preferred_element_type=_f32)
    k_ref[...] = jnp.dot(h, wk_ref[...], preferred_element_type=_f32)
    v_ref[...] = jnp.dot(h, wv_ref[...], preferred_element_type=_f32)


def _node_proj(z2, embp, wq, wk, wv):
    full = lambda a: pl.BlockSpec(a.shape, lambda i: (0,) * a.ndim)
    return pl.pallas_call(
        _node_proj_body,
        grid=(GN,),
        in_specs=[pl.BlockSpec((BN, 1), lambda i: (i, 0)),
                  full(embp), full(wq), full(wk), full(wv)],
        out_specs=[pl.BlockSpec((BN, F), lambda i: (i, 0))] * 4,
        out_shape=[jax.ShapeDtypeStruct((NPAD, F), _f32)] * 4,
    )(z2, embp, wq, wk, wv)


def _edge_geom(dr):
    r2 = jnp.sum(dr * dr, axis=1, keepdims=True)
    r = jnp.sqrt(r2 + 1e-12)
    rc = jnp.minimum(r, CUTOFF)
    cut = 0.5 * (jnp.cos(jnp.pi * rc / CUTOFF) + 1.0)
    cut = cut * (r < CUTOFF).astype(_f32)
    centers = (lax.broadcasted_iota(jnp.int32, (1, K), 1).astype(_f32)
               * (CUTOFF / (K - 1)))
    width = CUTOFF / K
    rbf = jnp.exp(-0.5 * ((r - centers) / width) ** 2)
    return r, cut, rbf


def _edge_feat_body(dr_ref, qg_ref, kg_ref, vg_ref,
                    fw1_ref, fb1_ref, fw2_ref, fb2_ref, msg_ref):
    dr = dr_ref[...]
    _, cut, rbf = _edge_geom(dr)
    s1 = _silu(jnp.dot(rbf, fw1_ref[...], preferred_element_type=_f32)
               + fb1_ref[...])
    wf = jnp.dot(s1, fw2_ref[...], preferred_element_type=_f32) + fb2_ref[...]
    t = qg_ref[...] * kg_ref[...] * wf * (cut * (1.0 / np.sqrt(DH)))
    hr = lax.broadcasted_iota(jnp.int32, (F, F), 0) // DH
    hc = lax.broadcasted_iota(jnp.int32, (F, F), 1) // DH
    s_mat = (hr == hc).astype(_f32)
    alpha = jnp.dot(t, s_mat, preferred_element_type=_f32)
    msg_ref[...] = alpha * vg_ref[...]


def _edge_feat(drp, qg, kg, vg, fw1, fb1, fw2, fb2):
    full = lambda a: pl.BlockSpec(a.shape, lambda i: (0,) * a.ndim)
    eb = pl.BlockSpec((BE, F), lambda i: (i, 0))
    return pl.pallas_call(
        _edge_feat_body,
        grid=(GE,),
        in_specs=[pl.BlockSpec((BE, 3), lambda i: (i, 0)), eb, eb, eb,
                  full(fw1), full(fb1), full(fw2), full(fb2)],
        out_specs=eb,
        out_shape=jax.ShapeDtypeStruct((EPAD, F), _f32),
    )(drp, qg, kg, vg, fw1, fb1, fw2, fb2)


def _node_upd_body(h_ref, a0_ref, a1_ref, wo_ref, wg_ref, h2_ref, g16_ref):
    agg = a0_ref[...] + a1_ref[...]
    h2 = h_ref[...] + jnp.dot(agg, wo_ref[...], preferred_element_type=_f32)
    h2_ref[...] = h2
    g16_ref[...] = jnp.dot(h2, wg_ref[...], preferred_element_type=_f32)


def _node_upd(h, aggf, wo, wg16):
    full = lambda a: pl.BlockSpec(a.shape, lambda i: (0,) * a.ndim)
    nb = pl.BlockSpec((BN, F), lambda i: (i, 0))
    return pl.pallas_call(
        _node_upd_body,
        grid=(GN,),
        in_specs=[nb, nb, pl.BlockSpec((BN, F), lambda i: (i + GN, 0)),
                  full(wo), full(wg16)],
        out_specs=[nb, nb],
        out_shape=[jax.ShapeDtypeStruct((NPAD, F), _f32),
                   jax.ShapeDtypeStruct((NPAD, F), _f32)],
    )(h, aggf, aggf, wo, wg16)


def _edge_geo_body(dr_ref, gi_ref, gj_ref,
                   gw1_ref, gb1_ref, gw2_ref, gb2_ref, ctr_ref):
    dr = dr_ref[...]
    r, cut, rbf = _edge_geom(dr)
    u = dr / r
    s1 = _silu(jnp.dot(rbf, gw1_ref[...], preferred_element_type=_f32)
               + gb1_ref[...])
    wg = jnp.dot(s1, gw2_ref[...], preferred_element_type=_f32) + gb2_ref[...]
    gi = gi_ref[...]
    gj = gj_ref[...]
    ag = gi[:, 0:3] * wg * gj[:, 4:7] * cut
    c16 = lax.broadcasted_iota(jnp.int32, (1, F), 1)
    agm = (ag[:, 0:1] * (c16 < 3).astype(_f32)
           + ag[:, 1:2] * ((c16 >= 3) & (c16 < 8)).astype(_f32)
           + ag[:, 2:3] * ((c16 >= 8) & (c16 < 15)).astype(_f32))
    x, y, z = u[:, 0:1], u[:, 1:2], u[:, 2:3]
    xx, yy, zz = x * x, y * y, z * z
    terms = [x, y, z,
             x * y, y * z, 3.0 * zz - 1.0, x * z, xx - yy,
             y * (3.0 * xx - yy), x * y * z, y * (5.0 * zz - 1.0),
             z * (5.0 * zz - 3.0), x * (5.0 * zz - 1.0),
             z * (xx - yy), x * (xx - 3.0 * yy)]
    sph = terms[0] * (c16 == 0).astype(_f32)
    for ti in range(1, 15):
        sph = sph + terms[ti] * (c16 == ti).astype(_f32)
    ctr_ref[...] = agm * sph


def _edge_geo(drp, gi, gj, gw1, gb1, gw2, gb2):
    full = lambda a: pl.BlockSpec(a.shape, lambda i: (0,) * a.ndim)
    eb = pl.BlockSpec((BE, F), lambda i: (i, 0))
    return pl.pallas_call(
        _edge_geo_body,
        grid=(GE,),
        in_specs=[pl.BlockSpec((BE, 3), lambda i: (i, 0)), eb, eb,
                  full(gw1), full(gb1), full(gw2), full(gb2)],
        out_specs=eb,
        out_shape=jax.ShapeDtypeStruct((EPAD, F), _f32),
    )(drp, gi, gj, gw1, gb1, gw2, gb2)


def _node_final_body(h2_ref, c0_ref, c1_ref, z_ref, w1h_ref, w1d_ref, b1_ref,
                     w2h_ref, b2h_ref, row1_ref, rob1_ref, row2_ref, rob2_ref,
                     ss_ref, out_ref):
    chi = c0_ref[...] + c1_ref[...]
    d1 = jnp.sum(chi[:, 0:3] ** 2, axis=1, keepdims=True)
    d2 = jnp.sum(chi[:, 3:8] ** 2, axis=1, keepdims=True)
    d3 = jnp.sum(chi[:, 8:15] ** 2, axis=1, keepdims=True)
    h2 = h2_ref[...]
    w1d = w1d_ref[...]
    pre = (jnp.dot(h2, w1h_ref[...], preferred_element_type=_f32)
           + d1 * w1d[0:1, :] + d2 * w1d[1:2, :] + d3 * w1d[2:3, :]
           + b1_ref[...])
    s = _silu(pre)
    h3 = h2 + jnp.dot(s, w2h_ref[...], preferred_element_type=_f32) + b2h_ref[...]
    t = _silu(jnp.dot(h3, row1_ref[...], preferred_element_type=_f32)
              + rob1_ref[...])
    e = jnp.dot(t, row2_ref[...], preferred_element_type=_f32) + rob2_ref[...]
    z = z_ref[...]
    nm = (z > 0).astype(_f32)
    oh = (lax.broadcasted_iota(jnp.int32, (BN, 128), 1) == z).astype(_f32)
    ss = jnp.dot(oh, ss_ref[...], preferred_element_type=_f32)
    out_ref[...] = ss[:, 0:1] * (e * nm) + ss[:, 1:2]


def _node_final(h2, chif, z2, w1h, w1d, b1, w2h, b2h,
                row1, rob1, row2, rob2, sspack):
    full = lambda a: pl.BlockSpec(a.shape, lambda i: (0,) * a.ndim)
    nb = pl.BlockSpec((BN, F), lambda i: (i, 0))
    return pl.pallas_call(
        _node_final_body,
        grid=(GN,),
        in_specs=[nb,
                  nb, pl.BlockSpec((BN, F), lambda i: (i + GN, 0)),
                  pl.BlockSpec((BN, 1), lambda i: (i, 0)),
                  full(w1h), full(w1d), full(b1), full(w2h), full(b2h),
                  full(row1), full(rob1), full(row2), full(rob2),
                  full(sspack)],
        out_specs=pl.BlockSpec((BN, 1), lambda i: (i, 0)),
        out_shape=jax.ShapeDtypeStruct((NPAD, 1), _f32),
    )(h2, chif, chif, z2, w1h, w1d, b1, w2h, b2h,
      row1, rob1, row2, rob2, sspack)


def _sc_gather(tables, idxs, width=F):
    nt = len(tables)
    scratch = ([pltpu.VMEM((CH,), jnp.int32) for _ in range(nt)]
               + [pltpu.VMEM((CH, width), _f32) for _ in range(nt)]
               + [pltpu.SemaphoreType.DMA for _ in range(nt)])

    @functools.partial(
        pl.kernel,
        out_type=[jax.ShapeDtypeStruct((EPAD, width), _f32)] * nt,
        mesh=_sc_mesh(),
        scratch_types=scratch,
    )
    def body(*refs):
        tabs = refs[:nt]
        idx_hbm = refs[nt:2 * nt]
        outs = refs[2 * nt:3 * nt]
        idx_v = refs[3 * nt:4 * nt]
        bufs = refs[4 * nt:5 * nt]
        sems = refs[5 * nt:6 * nt]
        wid = lax.axis_index("s") * NC + lax.axis_index("c")
        base = wid * EW

        def chunk(c, carry):
            off = base + c * CH
            for t in range(nt):
                pltpu.sync_copy(idx_hbm[t].at[pl.ds(off, CH)], idx_v[t])
            descs = [pltpu.async_copy(tabs[t].at[idx_v[t]], bufs[t], sems[t])
                     for t in range(nt)]
            for t in range(nt):
                descs[t].wait()
                pltpu.sync_copy(bufs[t], outs[t].at[pl.ds(off, CH)])
            return carry

        lax.fori_loop(0, NCHUNK, chunk, 0)

    return body(*tables, *idxs)


def _sc_scatter_add(vals, iidx, width=F):
    @functools.partial(
        pl.kernel,
        out_type=jax.ShapeDtypeStruct((NC * NPAD, width), _f32),
        mesh=_sc_mesh(),
        scratch_types=[
            pltpu.VMEM((CH,), jnp.int32),
            pltpu.VMEM((CH, width), _f32),
            pltpu.VMEM_SHARED((NPAD, width), _f32),
        ],
    )
    def body(val_hbm, ii_hbm, out_hbm, ii_v, vb, acc):
        c = lax.axis_index("c")
        s = lax.axis_index("s")
        wid = s * NC + c
        base = wid * EW

        def zrow(r, carry):
            for cc in range(width // 16):
                vb[r, pl.ds(cc * 16, 16)] = jnp.zeros((16,), _f32)
            return carry
        lax.fori_loop(0, CH, zrow, 0)

        def zacc(kk, carry):
            pltpu.sync_copy(vb, acc.at[pl.ds(s * ROWS_N + kk * CH, CH)])
            return carry
        lax.fori_loop(0, ROWS_N // CH, zacc, 0)
        plsc.subcore_barrier()

        def chunk(cc, carry):
            off = base + cc * CH
            pltpu.sync_copy(ii_hbm.at[pl.ds(off, CH)], ii_v)
            pltpu.sync_copy(val_hbm.at[pl.ds(off, CH)], vb)
            pltpu.sync_copy(vb, acc.at[ii_v], add=True)
            return carry
        lax.fori_loop(0, NCHUNK, chunk, 0)
        plsc.subcore_barrier()

        pltpu.sync_copy(acc.at[pl.ds(s * ROWS_N, ROWS_N)],
                        out_hbm.at[pl.ds(c * NPAD + s * ROWS_N, ROWS_N)])

    return body(vals, iidx)


def kernel(dr_vec, Z, idx, params):
    p = params
    pad_dr = jnp.broadcast_to(
        jnp.array([2.0 * CUTOFF, 0.0, 0.0], _f32), (EPAD - E, 3))
    drp = jnp.concatenate([dr_vec, pad_dr], axis=0)
    iidx = jnp.concatenate([idx[0], jnp.zeros((EPAD - E,), jnp.int32)])
    jidx = jnp.concatenate([idx[1], jnp.zeros((EPAD - E,), jnp.int32)])
    z2 = jnp.pad(Z, (0, NPAD - N)).reshape(NPAD, 1)

    embp = jnp.zeros((128, F), _f32).at[:119].set(p['embed'])
    wg16 = (jnp.zeros((F, F), _f32)
            .at[:, 0:3].set(p['Wqg']).at[:, 4:7].set(p['Wkg']))
    w1h = p['int_W1'][:F]
    w1d = jnp.zeros((8, F), _f32).at[0:3].set(p['int_W1'][F:])
    b1 = p['int_b1'].reshape(1, F)
    w2h = p['int_W2'][:, :F]
    b2h = p['int_b2'][:F].reshape(1, F)
    fb1 = p['filt_b1'].reshape(1, K)
    fb2 = p['filt_b2'].reshape(1, F)
    gb1 = p['gfilt_b1'].reshape(1, K)
    gb2 = p['gfilt_b2'].reshape(1, 3)
    rob1 = p['ro_b1'].reshape(1, F // 2)
    rob2 = p['ro_b2'].reshape(1, 1)
    sspack = (jnp.zeros((128, 8), _f32)
              .at[:119, 0:1].set(p['scale']).at[:119, 1:2].set(p['shift']))

    h, q_t, k_t, v_t = _node_proj(z2, embp, p['Wq'], p['Wk'], p['Wv'])
    qg, kg, vg = _sc_gather([q_t, k_t, v_t], [iidx, jidx, jidx])
    msg = _edge_feat(drp, qg, kg, vg,
                     p['filt_W1'], fb1, p['filt_W2'], fb2)
    aggf = _sc_scatter_add(msg, iidx)
    h2, g16 = _node_upd(h, aggf, p['Wo'], wg16)
    gi, gj = _sc_gather([g16, g16], [iidx, jidx])
    ctr = _edge_geo(drp, gi, gj, p['gfilt_W1'], gb1, p['gfilt_W2'], gb2)
    chif = _sc_scatter_add(ctr, iidx)
    out = _node_final(h2, chif, z2, w1h, w1d, b1, w2h, b2h,
                      p['ro_W1'], rob1, p['ro_W2'], rob2, sspack)
    return out[:N]
